# baseline (speedup 1.0000x reference)
import functools

import jax
import jax.numpy as jnp
from jax import lax
from jax.experimental import pallas as pl
from jax.experimental.pallas import tpu as pltpu
from jax.experimental.pallas import tpu_sc as plsc

R = 20000
D = 1024
K = 117
KP = 128
TOPK = 100
THRESH = 0.05
BR = 4000

NW = 32
SHARD = R * KP // NW
WIN = 16000
NWIN = SHARD // WIN
NBINS = 576
BIN0 = 15692
BUFW = 48
NPRE = 256
CAND = NPRE + NW * BUFW
CROWS = CAND // 128
BIGI = 0x3FFFFFFF
SG = 40


def _score_body(x_ref, wt_ref, bias_ref, bs_ref, out_ref):
    logits = jnp.dot(x_ref[...], wt_ref[...],
                     preferred_element_type=jnp.float32)
    s = jax.nn.sigmoid(logits + bias_ref[...])
    v = s * jnp.transpose(bs_ref[0], (1, 0))
    out_ref[...] = jnp.where(v > THRESH, v, 0.0)


def _masked_scores(x, W, b, box_scores):
    wt = jnp.zeros((D, KP), jnp.float32).at[:, :K].set(W.T)
    bias = jnp.full((1, KP), -1e30, jnp.float32).at[0, :K].set(b)
    bs = box_scores.reshape(R // BR, 1, BR)
    return pl.pallas_call(
        _score_body,
        grid=(R // BR,),
        in_specs=[
            pl.BlockSpec((BR, D), lambda i: (i, 0)),
            pl.BlockSpec((D, KP), lambda i: (0, 0)),
            pl.BlockSpec((1, KP), lambda i: (0, 0)),
            pl.BlockSpec((1, 1, BR), lambda i: (i, 0, 0)),
        ],
        out_specs=pl.BlockSpec((BR, KP), lambda i: (i, 0)),
        out_shape=jax.ShapeDtypeStruct((R, KP), jnp.float32),
    )(x, wt, bias, bs)


def _worker_id():
    return lax.axis_index("s") * 2 + lax.axis_index("c")


def _bin_of(v):
    bits = lax.bitcast_convert_type(v, jnp.int32)
    return jnp.maximum(lax.shift_right_logical(bits, 16), BIN0) - BIN0


def _sc_hist(scores_flat):
    mesh = plsc.VectorSubcoreMesh(core_axis_name="c", subcore_axis_name="s")

    @functools.partial(
        pl.kernel, mesh=mesh,
        compiler_params=pltpu.CompilerParams(needs_layout_passes=False),
        out_type=jax.ShapeDtypeStruct((NW * NBINS,), jnp.int32),
        scratch_types=[
            pltpu.VMEM((WIN,), jnp.float32),
            pltpu.VMEM((WIN,), jnp.float32),
            pltpu.VMEM((16 * NBINS,), jnp.int32),
            pltpu.VMEM((NBINS,), jnp.int32),
            pltpu.SemaphoreType.DMA,
            pltpu.SemaphoreType.DMA,
        ],
    )
    def hist_kernel(sc_hbm, hist_hbm, buf0, buf1, lhist, hsum, sem0, sem1):
        wid = _worker_id()
        base = wid * SHARD
        lane = lax.iota(jnp.int32, 16)
        zeros16 = jnp.zeros((16,), jnp.int32)
        ones16 = jnp.ones((16,), jnp.int32)

        @plsc.parallel_loop(0, 16 * NBINS // 16, unroll=4)
        def _(i):
            lhist[pl.ds(i * 16, 16)] = zeros16

        bufs = (buf0, buf1)
        sems = (sem0, sem1)
        cps = [pltpu.async_copy(sc_hbm.at[pl.ds(base, WIN)], buf0, sem0)]

        def proc(buf):
            @plsc.parallel_loop(0, WIN // 16, unroll=8)
            def _(i):
                v = buf[pl.ds(i * 16, 16)]
                b = _bin_of(v)
                plsc.addupdate_scatter(lhist, [b * 16 + lane], ones16,
                                       mask=v > THRESH)

        for w in range(NWIN):
            if w + 1 < NWIN:
                cps.append(pltpu.async_copy(
                    sc_hbm.at[pl.ds(base + (w + 1) * WIN, WIN)],
                    bufs[(w + 1) % 2], sems[(w + 1) % 2]))
            cps[w].wait()
            proc(bufs[w % 2])

        @plsc.parallel_loop(0, NBINS // 16, unroll=2)
        def _(c):
            bin16 = (c * 16 + lane) * 16
            acc = zeros16
            for l in range(16):
                acc = acc + plsc.load_gather(lhist, [bin16 + ((lane + l) & 15)])
            hsum[pl.ds(c * 16, 16)] = acc
        pltpu.sync_copy(hsum, hist_hbm.at[pl.ds(wid * NBINS, NBINS)])

    return hist_kernel(scores_flat)


def _sc_compact(scores_flat, hists):
    mesh = plsc.VectorSubcoreMesh(core_axis_name="c", subcore_axis_name="s")

    @functools.partial(
        pl.kernel, mesh=mesh,
        compiler_params=pltpu.CompilerParams(needs_layout_passes=False),
        out_type=(jax.ShapeDtypeStruct((CAND,), jnp.float32),
                  jax.ShapeDtypeStruct((CAND,), jnp.int32)),
        scratch_types=[
            pltpu.VMEM((WIN,), jnp.float32),
            pltpu.VMEM((WIN,), jnp.float32),
            pltpu.VMEM((NW * NBINS,), jnp.int32),
            pltpu.VMEM((NBINS,), jnp.int32),
            pltpu.VMEM((BUFW,), jnp.float32),
            pltpu.VMEM((BUFW,), jnp.int32),
            pltpu.VMEM((NPRE,), jnp.float32),
            pltpu.VMEM((NPRE,), jnp.int32),
            pltpu.SemaphoreType.DMA,
            pltpu.SemaphoreType.DMA,
        ],
    )
    def compact_kernel(sc_hbm, hist_hbm, cval_hbm, cidx_hbm,
                       buf0, buf1, hrows, ghist, cval, cidx, pval, pidx,
                       sem0, sem1):
        wid = _worker_id()
        base = wid * SHARD
        lane = lax.iota(jnp.int32, 16)

        pltpu.sync_copy(hist_hbm, hrows)

        @plsc.parallel_loop(0, NBINS // 16, unroll=2)
        def _(c):
            acc = hrows[pl.ds(c * 16, 16)]
            for wk in range(1, NW):
                acc = acc + hrows[pl.ds(wk * NBINS + c * 16, 16)]
            ghist[pl.ds(c * 16, 16)] = acc

        def cut(c, carry):
            running, bstar = carry
            c2 = NBINS // 16 - 1 - c
            chunk = ghist[pl.ds(c2 * 16, 16)]
            rchunk = lax.rev(chunk, (0,))
            cs = plsc.cumsum(rchunk) + running
            bvals = c2 * 16 + 15 - lane
            bm = jnp.max(jnp.where(cs >= TOPK, bvals, -1))
            return running + jnp.sum(chunk), jnp.maximum(bstar, bm)
        _, bstar = lax.fori_loop(0, NBINS // 16, cut,
                                 (jnp.int32(0), jnp.int32(-1)))
        cutoff = lax.bitcast_convert_type(
            lax.shift_left(bstar + BIN0, jnp.int32(16)), jnp.float32)

        negs = jnp.full((16,), -1.0, jnp.float32)
        bigs = jnp.full((16,), BIGI, jnp.int32)

        def clr(i, _):
            cval[pl.ds(i * 16, 16)] = negs
            cidx[pl.ds(i * 16, 16)] = bigs
            return 0
        lax.fori_loop(0, BUFW // 16, clr, 0)

        @pl.when(wid == 0)
        def _():
            pltpu.async_copy(sc_hbm.at[pl.ds(0, NPRE)], pval, sem0).wait()
            def pre(i, _):
                e = i * 16 + lane
                r = lax.shift_right_logical(e, 7)
                kk = e & (KP - 1)
                pos = r * K + kk
                v = pval[pl.ds(i * 16, 16)]
                m = kk < K
                plsc.store_scatter(pidx, [jnp.where(m, pos, 0)], pos, mask=m)
                plsc.store_scatter(pval, [jnp.where(m, pos, 0)], v, mask=m)
                return 0
            lax.fori_loop(0, NPRE // 16, pre, 0)
            cv = pval[pl.ds(224, 16)]
            ci = pidx[pl.ds(224, 16)]
            pval[pl.ds(224, 16)] = jnp.where(lane < 10, cv, negs)
            pidx[pl.ds(224, 16)] = jnp.where(lane < 10, ci, bigs)
            pval[pl.ds(240, 16)] = negs
            pidx[pl.ds(240, 16)] = bigs
            pltpu.sync_copy(pval, cval_hbm.at[pl.ds(0, NPRE)])
            pltpu.sync_copy(pidx, cidx_hbm.at[pl.ds(0, NPRE)])

        bufs = (buf0, buf1)
        sems = (sem0, sem1)
        cp1 = pltpu.async_copy(sc_hbm.at[pl.ds(base, WIN)], buf1, sem1)
        cps = [cp1]

        def slow(i, cnt, buf, woff):
            v = buf[pl.ds(i * 16, 16)]
            e = base + woff + i * 16 + lane
            sel = (v >= cutoff) & (e >= NPRE)
            mi = jnp.where(sel, 1, 0).astype(jnp.int32)
            pos = cnt + plsc.cumsum(mi) - 1
            ok = sel & (pos < BUFW)
            safe = jnp.where(ok, pos, 0)
            r = lax.shift_right_logical(e, 7)
            kk = e & (KP - 1)
            plsc.store_scatter(cval, [safe], v, mask=ok)
            plsc.store_scatter(cidx, [safe], r * K + kk, mask=ok)
            return cnt + jnp.sum(mi)

        def proc(buf, woff, cnt0):
            @plsc.parallel_loop(0, WIN // (16 * SG), carry=cnt0)
            def cnt1(g, cnt):
                mxs = []
                for j in range(SG):
                    mxs.append(buf[pl.ds((g * SG + j) * 16, 16)])
                while len(mxs) > 1:
                    mxs = [jnp.maximum(mxs[2 * t], mxs[2 * t + 1])
                           for t in range(len(mxs) // 2)] + (
                              [mxs[-1]] if len(mxs) % 2 else [])
                hit = jnp.max(mxs[0]) >= cutoff

                def do_slow(c):
                    return lax.fori_loop(
                        g * SG, g * SG + SG,
                        functools.partial(slow, buf=buf, woff=woff), c)
                return lax.cond(hit, do_slow, lambda c: c, cnt)
            return cnt1

        cnt = jnp.int32(0)
        for w in range(NWIN):
            if w + 1 < NWIN:
                cps.append(pltpu.async_copy(
                    sc_hbm.at[pl.ds(base + (w + 1) * WIN, WIN)],
                    bufs[w % 2], sems[w % 2]))
            cps[w].wait()
            cnt = proc(bufs[(w + 1) % 2], w * WIN, cnt)

        pltpu.sync_copy(cval, cval_hbm.at[pl.ds(NPRE + wid * BUFW, BUFW)])
        pltpu.sync_copy(cidx, cidx_hbm.at[pl.ds(NPRE + wid * BUFW, BUFW)])

    return compact_kernel(scores_flat, hists)


def _tc_top100(cval, cidx):
    def body(vall_ref, iall_ref, vcol_ref, icol_ref, oval_ref, oidx_ref):
        vall = vall_ref[...]
        iall = iall_ref[...]
        col = lax.broadcasted_iota(jnp.int32, (1, 128), 1)
        out_v = jnp.zeros((1, 128), jnp.float32)
        out_i = jnp.zeros((1, 128), jnp.int32)
        for ib in range(CROWS):
            vi = vcol_ref[:, ib:ib + 1]
            ii = icol_ref[:, ib:ib + 1]
            gt = (vall > vi) | ((vall == vi) & (iall < ii))
            rank = jnp.sum(gt.astype(jnp.int32), axis=1,
                           keepdims=True)
            onehot = rank == col
            out_v = out_v + jnp.sum(jnp.where(onehot, vi, 0.0),
                                    axis=0, keepdims=True)
            out_i = out_i + jnp.sum(jnp.where(onehot, ii, 0),
                                    axis=0, keepdims=True)
        oval_ref[...] = out_v
        oidx_ref[...] = out_i

    return pl.pallas_call(
        body,
        out_shape=(jax.ShapeDtypeStruct((1, 128), jnp.float32),
                   jax.ShapeDtypeStruct((1, 128), jnp.int32)),
    )(cval.reshape(1, CAND), cidx.reshape(1, CAND),
      cval.reshape(CROWS, 128).T, cidx.reshape(CROWS, 128).T)


def _sc_gather(table, idx_pad):
    mesh = plsc.VectorSubcoreMesh(core_axis_name="c", subcore_axis_name="s")
    NB = idx_pad.shape[0]

    @functools.partial(
        pl.kernel, mesh=mesh,
        compiler_params=pltpu.CompilerParams(needs_layout_passes=False),
        out_type=jax.ShapeDtypeStruct((NB, 128), jnp.float32),
        scratch_types=[
            pltpu.VMEM((NB,), jnp.int32),
            pltpu.VMEM((NB, 128), jnp.float32),
            pltpu.SemaphoreType.DMA,
        ],
    )
    def gather_kernel(tbl_hbm, idx_hbm, out_hbm, idx_v, rows_v, sem):
        wid = _worker_id()

        @pl.when(wid == 0)
        def _():
            pltpu.sync_copy(idx_hbm, idx_v)
            pltpu.async_copy(tbl_hbm.at[idx_v], rows_v, sem).wait()
            pltpu.sync_copy(rows_v, out_hbm)

    return gather_kernel(table, idx_pad)


def kernel(x, person_boxes, object_boxes, person_box_scores,
           object_box_scores, object_box_classes, W, b):
    bs = person_box_scores * object_box_scores
    scores = _masked_scores(x, W, b, bs)
    scores_flat = scores.reshape(-1)
    hists = _sc_hist(scores_flat)
    cval, cidx = _sc_compact(scores_flat, hists)
    oval, oidx = _tc_top100(cval, cidx)
    top_scores = oval[0, :TOPK]
    top_idx = oidx[0, :TOPK]
    pair_idx = top_idx // K
    action_classes = top_idx % K

    clsf = object_box_classes.astype(jnp.float32).reshape(R, 1)
    table = jnp.pad(
        jnp.concatenate([person_boxes, object_boxes, clsf], axis=1),
        ((0, 0), (0, 119)))
    idx_pad = jnp.pad(pair_idx, (0, 12)).astype(jnp.int32)
    rows = _sc_gather(table, idx_pad)
    sel_person_boxes = rows[:TOPK, 0:4]
    sel_object_boxes = rows[:TOPK, 4:8]
    sel_object_classes = rows[:TOPK, 8].astype(object_box_classes.dtype)
    return (top_scores, sel_person_boxes, sel_object_boxes,
            sel_object_classes, action_classes)


# speedup vs baseline: 1.0100x; 1.0100x over previous
import functools

import jax
import jax.numpy as jnp
from jax import lax
from jax.experimental import pallas as pl
from jax.experimental.pallas import tpu as pltpu
from jax.experimental.pallas import tpu_sc as plsc

R = 20000
D = 1024
K = 117
KP = 128
TOPK = 100
THRESH = 0.05
BR = 2000

NW = 32
SHARD = R * KP // NW
WIN = 16000
NWIN = SHARD // WIN
NBINS = 576
BIN0 = 15692
BUFW = 48
NPRE = 256
CAND = NPRE + NW * BUFW
CROWS = CAND // 128
BIGI = 0x3FFFFFFF
SG = 40


def _score_body(x_ref, wt_ref, bias_ref, bs_ref, out_ref):
    logits = jnp.dot(x_ref[...], wt_ref[...],
                     preferred_element_type=jnp.float32)
    s = jax.nn.sigmoid(logits + bias_ref[...])
    v = s * jnp.transpose(bs_ref[0], (1, 0))
    out_ref[...] = jnp.where(v > THRESH, v, 0.0)


def _masked_scores(x, W, b, box_scores):
    wt = jnp.zeros((D, KP), jnp.float32).at[:, :K].set(W.T)
    bias = jnp.full((1, KP), -1e30, jnp.float32).at[0, :K].set(b)
    bs = box_scores.reshape(R // BR, 1, BR)
    return pl.pallas_call(
        _score_body,
        grid=(R // BR,),
        in_specs=[
            pl.BlockSpec((BR, D), lambda i: (i, 0)),
            pl.BlockSpec((D, KP), lambda i: (0, 0)),
            pl.BlockSpec((1, KP), lambda i: (0, 0)),
            pl.BlockSpec((1, 1, BR), lambda i: (i, 0, 0)),
        ],
        out_specs=pl.BlockSpec((BR, KP), lambda i: (i, 0)),
        out_shape=jax.ShapeDtypeStruct((R, KP), jnp.float32),
    )(x, wt, bias, bs)


def _worker_id():
    return lax.axis_index("s") * 2 + lax.axis_index("c")


def _bin_of(v):
    bits = lax.bitcast_convert_type(v, jnp.int32)
    return jnp.maximum(lax.shift_right_logical(bits, 16), BIN0) - BIN0


def _sc_hist(scores_flat):
    mesh = plsc.VectorSubcoreMesh(core_axis_name="c", subcore_axis_name="s")

    @functools.partial(
        pl.kernel, mesh=mesh,
        compiler_params=pltpu.CompilerParams(needs_layout_passes=False),
        out_type=jax.ShapeDtypeStruct((NW * NBINS,), jnp.int32),
        scratch_types=[
            pltpu.VMEM((WIN,), jnp.float32),
            pltpu.VMEM((WIN,), jnp.float32),
            pltpu.VMEM((16 * NBINS,), jnp.int32),
            pltpu.VMEM((NBINS,), jnp.int32),
            pltpu.SemaphoreType.DMA,
            pltpu.SemaphoreType.DMA,
        ],
    )
    def hist_kernel(sc_hbm, hist_hbm, buf0, buf1, lhist, hsum, sem0, sem1):
        wid = _worker_id()
        base = wid * SHARD
        lane = lax.iota(jnp.int32, 16)
        zeros16 = jnp.zeros((16,), jnp.int32)
        ones16 = jnp.ones((16,), jnp.int32)

        @plsc.parallel_loop(0, 16 * NBINS // 16, unroll=4)
        def _(i):
            lhist[pl.ds(i * 16, 16)] = zeros16

        bufs = (buf0, buf1)
        sems = (sem0, sem1)
        cps = [pltpu.async_copy(sc_hbm.at[pl.ds(base, WIN)], buf0, sem0)]

        def proc(buf):
            @plsc.parallel_loop(0, WIN // 16, unroll=8)
            def _(i):
                v = buf[pl.ds(i * 16, 16)]
                b = _bin_of(v)
                plsc.addupdate_scatter(lhist, [b * 16 + lane], ones16,
                                       mask=v > THRESH)

        for w in range(NWIN):
            if w + 1 < NWIN:
                cps.append(pltpu.async_copy(
                    sc_hbm.at[pl.ds(base + (w + 1) * WIN, WIN)],
                    bufs[(w + 1) % 2], sems[(w + 1) % 2]))
            cps[w].wait()
            proc(bufs[w % 2])

        @plsc.parallel_loop(0, NBINS // 16, unroll=2)
        def _(c):
            bin16 = (c * 16 + lane) * 16
            acc = zeros16
            for l in range(16):
                acc = acc + plsc.load_gather(lhist, [bin16 + ((lane + l) & 15)])
            hsum[pl.ds(c * 16, 16)] = acc
        pltpu.sync_copy(hsum, hist_hbm.at[pl.ds(wid * NBINS, NBINS)])

    return hist_kernel(scores_flat)


def _sc_compact(scores_flat, hists):
    mesh = plsc.VectorSubcoreMesh(core_axis_name="c", subcore_axis_name="s")

    @functools.partial(
        pl.kernel, mesh=mesh,
        compiler_params=pltpu.CompilerParams(needs_layout_passes=False),
        out_type=(jax.ShapeDtypeStruct((CAND,), jnp.float32),
                  jax.ShapeDtypeStruct((CAND,), jnp.int32)),
        scratch_types=[
            pltpu.VMEM((WIN,), jnp.float32),
            pltpu.VMEM((WIN,), jnp.float32),
            pltpu.VMEM((NW * NBINS,), jnp.int32),
            pltpu.VMEM((NBINS,), jnp.int32),
            pltpu.VMEM((BUFW,), jnp.float32),
            pltpu.VMEM((BUFW,), jnp.int32),
            pltpu.VMEM((NPRE,), jnp.float32),
            pltpu.VMEM((NPRE,), jnp.int32),
            pltpu.SemaphoreType.DMA,
            pltpu.SemaphoreType.DMA,
        ],
    )
    def compact_kernel(sc_hbm, hist_hbm, cval_hbm, cidx_hbm,
                       buf0, buf1, hrows, ghist, cval, cidx, pval, pidx,
                       sem0, sem1):
        wid = _worker_id()
        base = wid * SHARD
        lane = lax.iota(jnp.int32, 16)

        pltpu.sync_copy(hist_hbm, hrows)

        @plsc.parallel_loop(0, NBINS // 16, unroll=2)
        def _(c):
            acc = hrows[pl.ds(c * 16, 16)]
            for wk in range(1, NW):
                acc = acc + hrows[pl.ds(wk * NBINS + c * 16, 16)]
            ghist[pl.ds(c * 16, 16)] = acc

        def cut(c, carry):
            running, bstar = carry
            c2 = NBINS // 16 - 1 - c
            chunk = ghist[pl.ds(c2 * 16, 16)]
            rchunk = lax.rev(chunk, (0,))
            cs = plsc.cumsum(rchunk) + running
            bvals = c2 * 16 + 15 - lane
            bm = jnp.max(jnp.where(cs >= TOPK, bvals, -1))
            return running + jnp.sum(chunk), jnp.maximum(bstar, bm)
        _, bstar = lax.fori_loop(0, NBINS // 16, cut,
                                 (jnp.int32(0), jnp.int32(-1)))
        cutoff = lax.bitcast_convert_type(
            lax.shift_left(bstar + BIN0, jnp.int32(16)), jnp.float32)

        negs = jnp.full((16,), -1.0, jnp.float32)
        bigs = jnp.full((16,), BIGI, jnp.int32)

        def clr(i, _):
            cval[pl.ds(i * 16, 16)] = negs
            cidx[pl.ds(i * 16, 16)] = bigs
            return 0
        lax.fori_loop(0, BUFW // 16, clr, 0)

        @pl.when(wid == 0)
        def _():
            pltpu.async_copy(sc_hbm.at[pl.ds(0, NPRE)], pval, sem0).wait()
            def pre(i, _):
                e = i * 16 + lane
                r = lax.shift_right_logical(e, 7)
                kk = e & (KP - 1)
                pos = r * K + kk
                v = pval[pl.ds(i * 16, 16)]
                m = kk < K
                plsc.store_scatter(pidx, [jnp.where(m, pos, 0)], pos, mask=m)
                plsc.store_scatter(pval, [jnp.where(m, pos, 0)], v, mask=m)
                return 0
            lax.fori_loop(0, NPRE // 16, pre, 0)
            cv = pval[pl.ds(224, 16)]
            ci = pidx[pl.ds(224, 16)]
            pval[pl.ds(224, 16)] = jnp.where(lane < 10, cv, negs)
            pidx[pl.ds(224, 16)] = jnp.where(lane < 10, ci, bigs)
            pval[pl.ds(240, 16)] = negs
            pidx[pl.ds(240, 16)] = bigs
            pltpu.sync_copy(pval, cval_hbm.at[pl.ds(0, NPRE)])
            pltpu.sync_copy(pidx, cidx_hbm.at[pl.ds(0, NPRE)])

        bufs = (buf0, buf1)
        sems = (sem0, sem1)
        cp1 = pltpu.async_copy(sc_hbm.at[pl.ds(base, WIN)], buf1, sem1)
        cps = [cp1]

        def slow(i, cnt, buf, woff):
            v = buf[pl.ds(i * 16, 16)]
            e = base + woff + i * 16 + lane
            sel = (v >= cutoff) & (e >= NPRE)
            mi = jnp.where(sel, 1, 0).astype(jnp.int32)
            pos = cnt + plsc.cumsum(mi) - 1
            ok = sel & (pos < BUFW)
            safe = jnp.where(ok, pos, 0)
            r = lax.shift_right_logical(e, 7)
            kk = e & (KP - 1)
            plsc.store_scatter(cval, [safe], v, mask=ok)
            plsc.store_scatter(cidx, [safe], r * K + kk, mask=ok)
            return cnt + jnp.sum(mi)

        def proc(buf, woff, cnt0):
            @plsc.parallel_loop(0, WIN // (16 * SG), carry=cnt0)
            def cnt1(g, cnt):
                mxs = []
                for j in range(SG):
                    mxs.append(buf[pl.ds((g * SG + j) * 16, 16)])
                while len(mxs) > 1:
                    mxs = [jnp.maximum(mxs[2 * t], mxs[2 * t + 1])
                           for t in range(len(mxs) // 2)] + (
                              [mxs[-1]] if len(mxs) % 2 else [])
                hit = jnp.max(mxs[0]) >= cutoff

                def do_slow(c):
                    return lax.fori_loop(
                        g * SG, g * SG + SG,
                        functools.partial(slow, buf=buf, woff=woff), c)
                return lax.cond(hit, do_slow, lambda c: c, cnt)
            return cnt1

        cnt = jnp.int32(0)
        for w in range(NWIN):
            if w + 1 < NWIN:
                cps.append(pltpu.async_copy(
                    sc_hbm.at[pl.ds(base + (w + 1) * WIN, WIN)],
                    bufs[w % 2], sems[w % 2]))
            cps[w].wait()
            cnt = proc(bufs[(w + 1) % 2], w * WIN, cnt)

        pltpu.sync_copy(cval, cval_hbm.at[pl.ds(NPRE + wid * BUFW, BUFW)])
        pltpu.sync_copy(cidx, cidx_hbm.at[pl.ds(NPRE + wid * BUFW, BUFW)])

    return compact_kernel(scores_flat, hists)


def _tc_top100(cval, cidx):
    def body(vall_ref, iall_ref, vcol_ref, icol_ref, oval_ref, oidx_ref):
        vall = vall_ref[...]
        iall = iall_ref[...]
        col = lax.broadcasted_iota(jnp.int32, (1, 128), 1)
        out_v = jnp.zeros((1, 128), jnp.float32)
        out_i = jnp.zeros((1, 128), jnp.int32)
        for ib in range(CROWS):
            vi = vcol_ref[:, ib:ib + 1]
            ii = icol_ref[:, ib:ib + 1]
            gt = (vall > vi) | ((vall == vi) & (iall < ii))
            rank = jnp.sum(gt.astype(jnp.int32), axis=1,
                           keepdims=True)
            onehot = rank == col
            out_v = out_v + jnp.sum(jnp.where(onehot, vi, 0.0),
                                    axis=0, keepdims=True)
            out_i = out_i + jnp.sum(jnp.where(onehot, ii, 0),
                                    axis=0, keepdims=True)
        oval_ref[...] = out_v
        oidx_ref[...] = out_i

    return pl.pallas_call(
        body,
        out_shape=(jax.ShapeDtypeStruct((1, 128), jnp.float32),
                   jax.ShapeDtypeStruct((1, 128), jnp.int32)),
    )(cval.reshape(1, CAND), cidx.reshape(1, CAND),
      cval.reshape(CROWS, 128).T, cidx.reshape(CROWS, 128).T)


def _sc_gather(table, idx_pad):
    mesh = plsc.VectorSubcoreMesh(core_axis_name="c", subcore_axis_name="s")
    NB = idx_pad.shape[0]

    @functools.partial(
        pl.kernel, mesh=mesh,
        compiler_params=pltpu.CompilerParams(needs_layout_passes=False),
        out_type=jax.ShapeDtypeStruct((NB, 128), jnp.float32),
        scratch_types=[
            pltpu.VMEM((NB,), jnp.int32),
            pltpu.VMEM((NB, 128), jnp.float32),
            pltpu.SemaphoreType.DMA,
        ],
    )
    def gather_kernel(tbl_hbm, idx_hbm, out_hbm, idx_v, rows_v, sem):
        wid = _worker_id()

        @pl.when(wid == 0)
        def _():
            pltpu.sync_copy(idx_hbm, idx_v)
            pltpu.async_copy(tbl_hbm.at[idx_v], rows_v, sem).wait()
            pltpu.sync_copy(rows_v, out_hbm)

    return gather_kernel(table, idx_pad)


def kernel(x, person_boxes, object_boxes, person_box_scores,
           object_box_scores, object_box_classes, W, b):
    bs = person_box_scores * object_box_scores
    scores = _masked_scores(x, W, b, bs)
    scores_flat = scores.reshape(-1)
    hists = _sc_hist(scores_flat)
    cval, cidx = _sc_compact(scores_flat, hists)
    oval, oidx = _tc_top100(cval, cidx)
    top_scores = oval[0, :TOPK]
    top_idx = oidx[0, :TOPK]
    pair_idx = top_idx // K
    action_classes = top_idx % K

    clsf = object_box_classes.astype(jnp.float32).reshape(R, 1)
    table = jnp.pad(
        jnp.concatenate([person_boxes, object_boxes, clsf], axis=1),
        ((0, 0), (0, 119)))
    idx_pad = jnp.pad(pair_idx, (0, 12)).astype(jnp.int32)
    rows = _sc_gather(table, idx_pad)
    sel_person_boxes = rows[:TOPK, 0:4]
    sel_object_boxes = rows[:TOPK, 4:8]
    sel_object_classes = rows[:TOPK, 8].astype(object_box_classes.dtype)
    return (top_scores, sel_person_boxes, sel_object_boxes,
            sel_object_classes, action_classes)


# speedup vs baseline: 1.0267x; 1.0165x over previous
import functools

import jax
import jax.numpy as jnp
from jax import lax
from jax.experimental import pallas as pl
from jax.experimental.pallas import tpu as pltpu
from jax.experimental.pallas import tpu_sc as plsc

R = 20000
D = 1024
K = 117
KP = 128
TOPK = 100
THRESH = 0.05
BR = 2000

NW = 32
SHARD = R * KP // NW
WIN = 16000
NWIN = SHARD // WIN
NBINS = 576
BIN0 = 15692
BUFW = 32
NPRE = 256
CAND = NPRE + NW * BUFW
CROWS = CAND // 128
BIGI = 0x3FFFFFFF
SG = 40


def _score_body(x_ref, wt_ref, bias_ref, bs_ref, out_ref):
    logits = jnp.dot(x_ref[...], wt_ref[...],
                     preferred_element_type=jnp.float32)
    s = jax.nn.sigmoid(logits + bias_ref[...])
    v = s * jnp.transpose(bs_ref[0], (1, 0))
    out_ref[...] = jnp.where(v > THRESH, v, 0.0)


def _masked_scores(x, W, b, box_scores):
    wt = jnp.zeros((D, KP), jnp.float32).at[:, :K].set(W.T)
    bias = jnp.full((1, KP), -1e30, jnp.float32).at[0, :K].set(b)
    bs = box_scores.reshape(R // BR, 1, BR)
    return pl.pallas_call(
        _score_body,
        grid=(R // BR,),
        in_specs=[
            pl.BlockSpec((BR, D), lambda i: (i, 0)),
            pl.BlockSpec((D, KP), lambda i: (0, 0)),
            pl.BlockSpec((1, KP), lambda i: (0, 0)),
            pl.BlockSpec((1, 1, BR), lambda i: (i, 0, 0)),
        ],
        out_specs=pl.BlockSpec((BR, KP), lambda i: (i, 0)),
        out_shape=jax.ShapeDtypeStruct((R, KP), jnp.float32),
    )(x, wt, bias, bs)


def _worker_id():
    return lax.axis_index("s") * 2 + lax.axis_index("c")


def _bin_of(v):
    bits = lax.bitcast_convert_type(v, jnp.int32)
    return jnp.maximum(lax.shift_right_logical(bits, 16), BIN0) - BIN0


def _sc_hist(scores_flat):
    mesh = plsc.VectorSubcoreMesh(core_axis_name="c", subcore_axis_name="s")

    @functools.partial(
        pl.kernel, mesh=mesh,
        compiler_params=pltpu.CompilerParams(needs_layout_passes=False),
        out_type=jax.ShapeDtypeStruct((NW * NBINS,), jnp.int32),
        scratch_types=[
            pltpu.VMEM((WIN,), jnp.float32),
            pltpu.VMEM((WIN,), jnp.float32),
            pltpu.VMEM((16 * NBINS,), jnp.int32),
            pltpu.VMEM((NBINS,), jnp.int32),
            pltpu.SemaphoreType.DMA,
            pltpu.SemaphoreType.DMA,
        ],
    )
    def hist_kernel(sc_hbm, hist_hbm, buf0, buf1, lhist, hsum, sem0, sem1):
        wid = _worker_id()
        base = wid * SHARD
        lane = lax.iota(jnp.int32, 16)
        zeros16 = jnp.zeros((16,), jnp.int32)
        ones16 = jnp.ones((16,), jnp.int32)

        @plsc.parallel_loop(0, 16 * NBINS // 16, unroll=4)
        def _(i):
            lhist[pl.ds(i * 16, 16)] = zeros16

        bufs = (buf0, buf1)
        sems = (sem0, sem1)
        cps = [pltpu.async_copy(sc_hbm.at[pl.ds(base, WIN)], buf0, sem0)]

        def proc(buf):
            @plsc.parallel_loop(0, WIN // 16, unroll=8)
            def _(i):
                v = buf[pl.ds(i * 16, 16)]
                b = _bin_of(v)
                plsc.addupdate_scatter(lhist, [b * 16 + lane], ones16,
                                       mask=v > THRESH)

        for w in range(NWIN):
            if w + 1 < NWIN:
                cps.append(pltpu.async_copy(
                    sc_hbm.at[pl.ds(base + (w + 1) * WIN, WIN)],
                    bufs[(w + 1) % 2], sems[(w + 1) % 2]))
            cps[w].wait()
            proc(bufs[w % 2])

        @plsc.parallel_loop(0, NBINS // 16, unroll=2)
        def _(c):
            bin16 = (c * 16 + lane) * 16
            acc = zeros16
            for l in range(16):
                acc = acc + plsc.load_gather(lhist, [bin16 + ((lane + l) & 15)])
            hsum[pl.ds(c * 16, 16)] = acc
        pltpu.sync_copy(hsum, hist_hbm.at[pl.ds(wid * NBINS, NBINS)])

    return hist_kernel(scores_flat)


def _sc_compact(scores_flat, hists):
    mesh = plsc.VectorSubcoreMesh(core_axis_name="c", subcore_axis_name="s")

    @functools.partial(
        pl.kernel, mesh=mesh,
        compiler_params=pltpu.CompilerParams(needs_layout_passes=False),
        out_type=(jax.ShapeDtypeStruct((CAND,), jnp.float32),
                  jax.ShapeDtypeStruct((CAND,), jnp.int32)),
        scratch_types=[
            pltpu.VMEM((WIN,), jnp.float32),
            pltpu.VMEM((WIN,), jnp.float32),
            pltpu.VMEM((NW * NBINS,), jnp.int32),
            pltpu.VMEM((NBINS,), jnp.int32),
            pltpu.VMEM((BUFW,), jnp.float32),
            pltpu.VMEM((BUFW,), jnp.int32),
            pltpu.VMEM((NPRE,), jnp.float32),
            pltpu.VMEM((NPRE,), jnp.int32),
            pltpu.SemaphoreType.DMA,
            pltpu.SemaphoreType.DMA,
        ],
    )
    def compact_kernel(sc_hbm, hist_hbm, cval_hbm, cidx_hbm,
                       buf0, buf1, hrows, ghist, cval, cidx, pval, pidx,
                       sem0, sem1):
        wid = _worker_id()
        base = wid * SHARD
        lane = lax.iota(jnp.int32, 16)

        pltpu.sync_copy(hist_hbm, hrows)

        @plsc.parallel_loop(0, NBINS // 16, unroll=2)
        def _(c):
            acc = hrows[pl.ds(c * 16, 16)]
            for wk in range(1, NW):
                acc = acc + hrows[pl.ds(wk * NBINS + c * 16, 16)]
            ghist[pl.ds(c * 16, 16)] = acc

        def cut(c, carry):
            running, bstar = carry
            c2 = NBINS // 16 - 1 - c
            chunk = ghist[pl.ds(c2 * 16, 16)]
            rchunk = lax.rev(chunk, (0,))
            cs = plsc.cumsum(rchunk) + running
            bvals = c2 * 16 + 15 - lane
            bm = jnp.max(jnp.where(cs >= TOPK, bvals, -1))
            return running + jnp.sum(chunk), jnp.maximum(bstar, bm)
        _, bstar = lax.fori_loop(0, NBINS // 16, cut,
                                 (jnp.int32(0), jnp.int32(-1)))
        cutoff = lax.bitcast_convert_type(
            lax.shift_left(bstar + BIN0, jnp.int32(16)), jnp.float32)

        negs = jnp.full((16,), -1.0, jnp.float32)
        bigs = jnp.full((16,), BIGI, jnp.int32)

        def clr(i, _):
            cval[pl.ds(i * 16, 16)] = negs
            cidx[pl.ds(i * 16, 16)] = bigs
            return 0
        lax.fori_loop(0, BUFW // 16, clr, 0)

        @pl.when(wid == 0)
        def _():
            pltpu.async_copy(sc_hbm.at[pl.ds(0, NPRE)], pval, sem0).wait()
            def pre(i, _):
                e = i * 16 + lane
                r = lax.shift_right_logical(e, 7)
                kk = e & (KP - 1)
                pos = r * K + kk
                v = pval[pl.ds(i * 16, 16)]
                m = kk < K
                plsc.store_scatter(pidx, [jnp.where(m, pos, 0)], pos, mask=m)
                plsc.store_scatter(pval, [jnp.where(m, pos, 0)], v, mask=m)
                return 0
            lax.fori_loop(0, NPRE // 16, pre, 0)
            cv = pval[pl.ds(224, 16)]
            ci = pidx[pl.ds(224, 16)]
            pval[pl.ds(224, 16)] = jnp.where(lane < 10, cv, negs)
            pidx[pl.ds(224, 16)] = jnp.where(lane < 10, ci, bigs)
            pval[pl.ds(240, 16)] = negs
            pidx[pl.ds(240, 16)] = bigs
            pltpu.sync_copy(pval, cval_hbm.at[pl.ds(0, NPRE)])
            pltpu.sync_copy(pidx, cidx_hbm.at[pl.ds(0, NPRE)])

        bufs = (buf0, buf1)
        sems = (sem0, sem1)
        cp1 = pltpu.async_copy(sc_hbm.at[pl.ds(base, WIN)], buf1, sem1)
        cps = [cp1]

        def slow(i, cnt, buf, woff):
            v = buf[pl.ds(i * 16, 16)]
            e = base + woff + i * 16 + lane
            sel = (v >= cutoff) & (e >= NPRE)
            mi = jnp.where(sel, 1, 0).astype(jnp.int32)
            pos = cnt + plsc.cumsum(mi) - 1
            ok = sel & (pos < BUFW)
            safe = jnp.where(ok, pos, 0)
            r = lax.shift_right_logical(e, 7)
            kk = e & (KP - 1)
            plsc.store_scatter(cval, [safe], v, mask=ok)
            plsc.store_scatter(cidx, [safe], r * K + kk, mask=ok)
            return cnt + jnp.sum(mi)

        def proc(buf, woff, cnt0):
            @plsc.parallel_loop(0, WIN // (16 * SG), carry=cnt0)
            def cnt1(g, cnt):
                mxs = []
                for j in range(SG):
                    mxs.append(buf[pl.ds((g * SG + j) * 16, 16)])
                while len(mxs) > 1:
                    mxs = [jnp.maximum(mxs[2 * t], mxs[2 * t + 1])
                           for t in range(len(mxs) // 2)] + (
                              [mxs[-1]] if len(mxs) % 2 else [])
                hit = jnp.max(mxs[0]) >= cutoff

                def do_slow(c):
                    return lax.fori_loop(
                        g * SG, g * SG + SG,
                        functools.partial(slow, buf=buf, woff=woff), c)
                return lax.cond(hit, do_slow, lambda c: c, cnt)
            return cnt1

        cnt = jnp.int32(0)
        for w in range(NWIN):
            if w + 1 < NWIN:
                cps.append(pltpu.async_copy(
                    sc_hbm.at[pl.ds(base + (w + 1) * WIN, WIN)],
                    bufs[w % 2], sems[w % 2]))
            cps[w].wait()
            cnt = proc(bufs[(w + 1) % 2], w * WIN, cnt)

        pltpu.sync_copy(cval, cval_hbm.at[pl.ds(NPRE + wid * BUFW, BUFW)])
        pltpu.sync_copy(cidx, cidx_hbm.at[pl.ds(NPRE + wid * BUFW, BUFW)])

    return compact_kernel(scores_flat, hists)


def _tc_top100(cval, cidx):
    def body(vall_ref, iall_ref, vcol_ref, icol_ref, oval_ref, oidx_ref):
        vall = vall_ref[...]
        iall = iall_ref[...]
        col = lax.broadcasted_iota(jnp.int32, (1, 128), 1)
        out_v = jnp.zeros((1, 128), jnp.float32)
        out_i = jnp.zeros((1, 128), jnp.int32)
        for ib in range(CROWS):
            vi = vcol_ref[:, ib:ib + 1]
            ii = icol_ref[:, ib:ib + 1]
            gt = (vall > vi) | ((vall == vi) & (iall < ii))
            rank = jnp.sum(gt.astype(jnp.int32), axis=1,
                           keepdims=True)
            onehot = rank == col
            out_v = out_v + jnp.sum(jnp.where(onehot, vi, 0.0),
                                    axis=0, keepdims=True)
            out_i = out_i + jnp.sum(jnp.where(onehot, ii, 0),
                                    axis=0, keepdims=True)
        oval_ref[...] = out_v
        oidx_ref[...] = out_i

    return pl.pallas_call(
        body,
        out_shape=(jax.ShapeDtypeStruct((1, 128), jnp.float32),
                   jax.ShapeDtypeStruct((1, 128), jnp.int32)),
    )(cval.reshape(1, CAND), cidx.reshape(1, CAND),
      cval.reshape(CROWS, 128).T, cidx.reshape(CROWS, 128).T)


def _sc_gather(table, idx_pad):
    mesh = plsc.VectorSubcoreMesh(core_axis_name="c", subcore_axis_name="s")
    NB = idx_pad.shape[0]

    @functools.partial(
        pl.kernel, mesh=mesh,
        compiler_params=pltpu.CompilerParams(needs_layout_passes=False),
        out_type=jax.ShapeDtypeStruct((NB, 128), jnp.float32),
        scratch_types=[
            pltpu.VMEM((NB,), jnp.int32),
            pltpu.VMEM((NB, 128), jnp.float32),
            pltpu.SemaphoreType.DMA,
        ],
    )
    def gather_kernel(tbl_hbm, idx_hbm, out_hbm, idx_v, rows_v, sem):
        wid = _worker_id()

        @pl.when(wid == 0)
        def _():
            pltpu.sync_copy(idx_hbm, idx_v)
            pltpu.async_copy(tbl_hbm.at[idx_v], rows_v, sem).wait()
            pltpu.sync_copy(rows_v, out_hbm)

    return gather_kernel(table, idx_pad)


def kernel(x, person_boxes, object_boxes, person_box_scores,
           object_box_scores, object_box_classes, W, b):
    bs = person_box_scores * object_box_scores
    scores = _masked_scores(x, W, b, bs)
    scores_flat = scores.reshape(-1)
    hists = _sc_hist(scores_flat)
    cval, cidx = _sc_compact(scores_flat, hists)
    oval, oidx = _tc_top100(cval, cidx)
    top_scores = oval[0, :TOPK]
    top_idx = oidx[0, :TOPK]
    pair_idx = top_idx // K
    action_classes = top_idx % K

    clsf = object_box_classes.astype(jnp.float32).reshape(R, 1)
    table = jnp.pad(
        jnp.concatenate([person_boxes, object_boxes, clsf], axis=1),
        ((0, 0), (0, 119)))
    idx_pad = oidx[0] // K
    rows = _sc_gather(table, idx_pad)
    sel_person_boxes = rows[:TOPK, 0:4]
    sel_object_boxes = rows[:TOPK, 4:8]
    sel_object_classes = rows[:TOPK, 8].astype(object_box_classes.dtype)
    return (top_scores, sel_person_boxes, sel_object_boxes,
            sel_object_classes, action_classes)


# speedup vs baseline: 1.0868x; 1.0586x over previous
import functools

import jax
import jax.numpy as jnp
from jax import lax
from jax.experimental import pallas as pl
from jax.experimental.pallas import tpu as pltpu
from jax.experimental.pallas import tpu_sc as plsc

R = 20000
D = 1024
K = 117
KP = 128
TOPK = 100
THRESH = 0.05
BR = 2000

NW = 32
SHARD = R * KP // NW
WIN = 16000
NWIN = SHARD // WIN
NBINS = 576
BIN0 = 15692
BUFW = 32
NPRE = 256
CAND = NPRE + NW * BUFW
CROWS = CAND // 128
BIGI = 0x3FFFFFFF
SG = 40


def _score_body(x_ref, wt_ref, bias_ref, bs_ref, out_ref):
    logits = jnp.dot(x_ref[...], wt_ref[...],
                     preferred_element_type=jnp.float32)
    s = jax.nn.sigmoid(logits + bias_ref[...])
    v = s * jnp.transpose(bs_ref[0], (1, 0))
    out_ref[...] = jnp.where(v > THRESH, v, 0.0)


def _masked_scores(x, W, b, box_scores):
    wt = jnp.zeros((D, KP), jnp.float32).at[:, :K].set(W.T)
    bias = jnp.full((1, KP), -1e30, jnp.float32).at[0, :K].set(b)
    bs = box_scores.reshape(R // BR, 1, BR)
    return pl.pallas_call(
        _score_body,
        grid=(R // BR,),
        in_specs=[
            pl.BlockSpec((BR, D), lambda i: (i, 0)),
            pl.BlockSpec((D, KP), lambda i: (0, 0)),
            pl.BlockSpec((1, KP), lambda i: (0, 0)),
            pl.BlockSpec((1, 1, BR), lambda i: (i, 0, 0)),
        ],
        out_specs=pl.BlockSpec((BR, KP), lambda i: (i, 0)),
        out_shape=jax.ShapeDtypeStruct((R, KP), jnp.float32),
    )(x, wt, bias, bs)


def _worker_id():
    return lax.axis_index("s") * 2 + lax.axis_index("c")


def _bin_of(v):
    bits = lax.bitcast_convert_type(v, jnp.int32)
    return jnp.maximum(lax.shift_right_logical(bits, 16), BIN0) - BIN0


def _sc_hist(scores_flat):
    mesh = plsc.VectorSubcoreMesh(core_axis_name="c", subcore_axis_name="s")

    @functools.partial(
        pl.kernel, mesh=mesh,
        compiler_params=pltpu.CompilerParams(needs_layout_passes=False),
        out_type=jax.ShapeDtypeStruct((NW * NBINS,), jnp.int32),
        scratch_types=[
            pltpu.VMEM((WIN,), jnp.float32),
            pltpu.VMEM((WIN,), jnp.float32),
            pltpu.VMEM((16 * NBINS,), jnp.int32),
            pltpu.VMEM((NBINS,), jnp.int32),
            pltpu.SemaphoreType.DMA,
            pltpu.SemaphoreType.DMA,
        ],
    )
    def hist_kernel(sc_hbm, hist_hbm, buf0, buf1, lhist, hsum, sem0, sem1):
        wid = _worker_id()
        base = wid * SHARD
        lane = lax.iota(jnp.int32, 16)
        zeros16 = jnp.zeros((16,), jnp.int32)
        ones16 = jnp.ones((16,), jnp.int32)

        @plsc.parallel_loop(0, 16 * NBINS // 16, unroll=4)
        def _(i):
            lhist[pl.ds(i * 16, 16)] = zeros16

        bufs = (buf0, buf1)
        sems = (sem0, sem1)
        cps = [pltpu.async_copy(sc_hbm.at[pl.ds(base, WIN)], buf0, sem0)]

        def proc(buf):
            @plsc.parallel_loop(0, WIN // 16, unroll=8)
            def _(i):
                v = buf[pl.ds(i * 16, 16)]
                b = _bin_of(v)
                plsc.addupdate_scatter(lhist, [b * 16 + lane], ones16,
                                       mask=v > THRESH)

        for w in range(NWIN):
            if w + 1 < NWIN:
                cps.append(pltpu.async_copy(
                    sc_hbm.at[pl.ds(base + (w + 1) * WIN, WIN)],
                    bufs[(w + 1) % 2], sems[(w + 1) % 2]))
            cps[w].wait()
            proc(bufs[w % 2])

        @plsc.parallel_loop(0, NBINS // 16, unroll=2)
        def _(c):
            bin16 = (c * 16 + lane) * 16
            acc = zeros16
            for l in range(16):
                acc = acc + plsc.load_gather(lhist, [bin16 + ((lane + l) & 15)])
            hsum[pl.ds(c * 16, 16)] = acc
        pltpu.sync_copy(hsum, hist_hbm.at[pl.ds(wid * NBINS, NBINS)])

    return hist_kernel(scores_flat)


def _sc_compact(scores_flat, hists):
    mesh = plsc.VectorSubcoreMesh(core_axis_name="c", subcore_axis_name="s")

    @functools.partial(
        pl.kernel, mesh=mesh,
        compiler_params=pltpu.CompilerParams(needs_layout_passes=False),
        out_type=(jax.ShapeDtypeStruct((CAND,), jnp.float32),
                  jax.ShapeDtypeStruct((CAND,), jnp.int32)),
        scratch_types=[
            pltpu.VMEM((WIN,), jnp.float32),
            pltpu.VMEM((WIN,), jnp.float32),
            pltpu.VMEM((NW * NBINS,), jnp.int32),
            pltpu.VMEM((NBINS,), jnp.int32),
            pltpu.VMEM((BUFW,), jnp.float32),
            pltpu.VMEM((BUFW,), jnp.int32),
            pltpu.VMEM((NPRE,), jnp.float32),
            pltpu.VMEM((NPRE,), jnp.int32),
            pltpu.SemaphoreType.DMA,
            pltpu.SemaphoreType.DMA,
        ],
    )
    def compact_kernel(sc_hbm, hist_hbm, cval_hbm, cidx_hbm,
                       buf0, buf1, hrows, ghist, cval, cidx, pval, pidx,
                       sem0, sem1):
        wid = _worker_id()
        base = wid * SHARD
        lane = lax.iota(jnp.int32, 16)

        pltpu.sync_copy(hist_hbm, hrows)

        @plsc.parallel_loop(0, NBINS // 16, unroll=2)
        def _(c):
            acc = hrows[pl.ds(c * 16, 16)]
            for wk in range(1, NW):
                acc = acc + hrows[pl.ds(wk * NBINS + c * 16, 16)]
            ghist[pl.ds(c * 16, 16)] = acc

        def cut(c, carry):
            running, bstar = carry
            c2 = NBINS // 16 - 1 - c
            chunk = ghist[pl.ds(c2 * 16, 16)]
            rchunk = lax.rev(chunk, (0,))
            cs = plsc.cumsum(rchunk) + running
            bvals = c2 * 16 + 15 - lane
            bm = jnp.max(jnp.where(cs >= TOPK, bvals, -1))
            return running + jnp.sum(chunk), jnp.maximum(bstar, bm)
        _, bstar = lax.fori_loop(0, NBINS // 16, cut,
                                 (jnp.int32(0), jnp.int32(-1)))
        cutoff = lax.bitcast_convert_type(
            lax.shift_left(bstar + BIN0, jnp.int32(16)), jnp.float32)

        negs = jnp.full((16,), -1.0, jnp.float32)
        bigs = jnp.full((16,), BIGI, jnp.int32)

        def clr(i, _):
            cval[pl.ds(i * 16, 16)] = negs
            cidx[pl.ds(i * 16, 16)] = bigs
            return 0
        lax.fori_loop(0, BUFW // 16, clr, 0)

        @pl.when(wid == 0)
        def _():
            pltpu.async_copy(sc_hbm.at[pl.ds(0, NPRE)], pval, sem0).wait()
            def pre(i, _):
                e = i * 16 + lane
                r = lax.shift_right_logical(e, 7)
                kk = e & (KP - 1)
                pos = r * K + kk
                v = pval[pl.ds(i * 16, 16)]
                m = kk < K
                plsc.store_scatter(pidx, [jnp.where(m, pos, 0)], pos, mask=m)
                plsc.store_scatter(pval, [jnp.where(m, pos, 0)], v, mask=m)
                return 0
            lax.fori_loop(0, NPRE // 16, pre, 0)
            cv = pval[pl.ds(224, 16)]
            ci = pidx[pl.ds(224, 16)]
            pval[pl.ds(224, 16)] = jnp.where(lane < 10, cv, negs)
            pidx[pl.ds(224, 16)] = jnp.where(lane < 10, ci, bigs)
            pval[pl.ds(240, 16)] = negs
            pidx[pl.ds(240, 16)] = bigs
            pltpu.sync_copy(pval, cval_hbm.at[pl.ds(0, NPRE)])
            pltpu.sync_copy(pidx, cidx_hbm.at[pl.ds(0, NPRE)])

        bufs = (buf0, buf1)
        sems = (sem0, sem1)
        cp1 = pltpu.async_copy(sc_hbm.at[pl.ds(base, WIN)], buf1, sem1)
        cps = [cp1]

        def slow(i, cnt, buf, woff):
            v = buf[pl.ds(i * 16, 16)]
            e = base + woff + i * 16 + lane
            sel = (v >= cutoff) & (e >= NPRE)
            mi = jnp.where(sel, 1, 0).astype(jnp.int32)
            pos = cnt + plsc.cumsum(mi) - 1
            ok = sel & (pos < BUFW)
            safe = jnp.where(ok, pos, 0)
            r = lax.shift_right_logical(e, 7)
            kk = e & (KP - 1)
            plsc.store_scatter(cval, [safe], v, mask=ok)
            plsc.store_scatter(cidx, [safe], r * K + kk, mask=ok)
            return cnt + jnp.sum(mi)

        def proc(buf, woff, cnt0):
            @plsc.parallel_loop(0, WIN // (16 * SG), carry=cnt0)
            def cnt1(g, cnt):
                mxs = []
                for j in range(SG):
                    mxs.append(buf[pl.ds((g * SG + j) * 16, 16)])
                while len(mxs) > 1:
                    mxs = [jnp.maximum(mxs[2 * t], mxs[2 * t + 1])
                           for t in range(len(mxs) // 2)] + (
                              [mxs[-1]] if len(mxs) % 2 else [])
                hit = jnp.max(mxs[0]) >= cutoff

                def do_slow(c):
                    return lax.fori_loop(
                        g * SG, g * SG + SG,
                        functools.partial(slow, buf=buf, woff=woff), c)
                return lax.cond(hit, do_slow, lambda c: c, cnt)
            return cnt1

        cnt = jnp.int32(0)
        for w in range(NWIN):
            if w + 1 < NWIN:
                cps.append(pltpu.async_copy(
                    sc_hbm.at[pl.ds(base + (w + 1) * WIN, WIN)],
                    bufs[w % 2], sems[w % 2]))
            cps[w].wait()
            cnt = proc(bufs[(w + 1) % 2], w * WIN, cnt)

        pltpu.sync_copy(cval, cval_hbm.at[pl.ds(NPRE + wid * BUFW, BUFW)])
        pltpu.sync_copy(cidx, cidx_hbm.at[pl.ds(NPRE + wid * BUFW, BUFW)])

    return compact_kernel(scores_flat, hists)


def _tc_top100(cval, cidx):
    def body(vrow_ref, irow_ref, oval_ref, oidx_ref):
        vrows = vrow_ref[...]
        irows = irow_ref[...]
        vall = vrows.reshape(1, CAND)
        iall = irows.reshape(1, CAND)
        vcol = jnp.transpose(vrows, (1, 0))
        icol = jnp.transpose(irows, (1, 0))
        col = lax.broadcasted_iota(jnp.int32, (1, 128), 1)
        out_v = jnp.zeros((1, 128), jnp.float32)
        out_i = jnp.zeros((1, 128), jnp.int32)
        for ib in range(CROWS):
            vi = vcol[:, ib:ib + 1]
            ii = icol[:, ib:ib + 1]
            gt = (vall > vi) | ((vall == vi) & (iall < ii))
            rank = jnp.sum(gt.astype(jnp.int32), axis=1,
                           keepdims=True)
            onehot = rank == col
            out_v = out_v + jnp.sum(jnp.where(onehot, vi, 0.0),
                                    axis=0, keepdims=True)
            out_i = out_i + jnp.sum(jnp.where(onehot, ii, 0),
                                    axis=0, keepdims=True)
        oval_ref[...] = out_v
        oidx_ref[...] = out_i

    return pl.pallas_call(
        body,
        out_shape=(jax.ShapeDtypeStruct((1, 128), jnp.float32),
                   jax.ShapeDtypeStruct((1, 128), jnp.int32)),
    )(cval.reshape(CROWS, 128), cidx.reshape(CROWS, 128))


def _sc_gather(table, idx_pad):
    mesh = plsc.VectorSubcoreMesh(core_axis_name="c", subcore_axis_name="s")
    NB = idx_pad.shape[0]

    @functools.partial(
        pl.kernel, mesh=mesh,
        compiler_params=pltpu.CompilerParams(needs_layout_passes=False),
        out_type=jax.ShapeDtypeStruct((NB, 128), jnp.float32),
        scratch_types=[
            pltpu.VMEM((NB,), jnp.int32),
            pltpu.VMEM((NB, 128), jnp.float32),
            pltpu.SemaphoreType.DMA,
        ],
    )
    def gather_kernel(tbl_hbm, idx_hbm, out_hbm, idx_v, rows_v, sem):
        wid = _worker_id()

        @pl.when(wid == 0)
        def _():
            pltpu.sync_copy(idx_hbm, idx_v)
            pltpu.async_copy(tbl_hbm.at[idx_v], rows_v, sem).wait()
            pltpu.sync_copy(rows_v, out_hbm)

    return gather_kernel(table, idx_pad)


def kernel(x, person_boxes, object_boxes, person_box_scores,
           object_box_scores, object_box_classes, W, b):
    bs = person_box_scores * object_box_scores
    scores = _masked_scores(x, W, b, bs)
    scores_flat = scores.reshape(-1)
    hists = _sc_hist(scores_flat)
    cval, cidx = _sc_compact(scores_flat, hists)
    oval, oidx = _tc_top100(cval, cidx)
    top_scores = oval[0, :TOPK]
    top_idx = oidx[0, :TOPK]
    pair_idx = top_idx // K
    action_classes = top_idx % K

    clsf = object_box_classes.astype(jnp.float32).reshape(R, 1)
    table = jnp.pad(
        jnp.concatenate([person_boxes, object_boxes, clsf], axis=1),
        ((0, 0), (0, 119)))
    idx_pad = oidx[0] // K
    rows = _sc_gather(table, idx_pad)
    sel_person_boxes = rows[:TOPK, 0:4]
    sel_object_boxes = rows[:TOPK, 4:8]
    sel_object_classes = rows[:TOPK, 8].astype(object_box_classes.dtype)
    return (top_scores, sel_person_boxes, sel_object_boxes,
            sel_object_classes, action_classes)
